# Initial kernel scaffold; baseline (speedup 1.0000x reference)
import jax
import jax.numpy as jnp
from jax.experimental import pallas as pl


def kernel(coords, pairs, box, charges, prefac, cutoff):
    raise NotImplementedError("write your pallas kernel here")


# speedup vs baseline: 18.5595x; 18.5595x over previous
import functools

import jax
import jax.numpy as jnp
from jax import lax
from jax.experimental import pallas as pl
from jax.experimental.pallas import tpu as pltpu
from jax.experimental.pallas import tpu_sc as plsc

_N = 100000
_E = 6400000
_NC = 2
_NS = 16
_NW = _NC * _NS
_L = 16

_IDXW = 128
_IDX_ROWS = 2 * _E // _IDXW
_ROWS_PER_W = _IDX_ROWS // _NW
_R = 25
_CHUNKS = _ROWS_PER_W // _R
_ENTRIES = _R * _IDXW
_STEPS = _ENTRIES // (2 * _L)

_MESH = plsc.VectorSubcoreMesh(core_axis_name="c", subcore_axis_name="s")


@functools.partial(
    pl.kernel,
    out_type=jax.ShapeDtypeStruct((_NW * _L,), jnp.float32),
    mesh=_MESH,
    compiler_params=pltpu.CompilerParams(needs_layout_passes=False,
                                         use_tc_tiling_on_sc=False),
    scratch_types=[
        pltpu.VMEM((_R, _IDXW), jnp.int32),
        pltpu.VMEM((_ENTRIES, 8), jnp.float32),
        pltpu.VMEM((20, 16), jnp.float32),
        pltpu.VMEM((_L,), jnp.float32),
        pltpu.SemaphoreType.DMA,
    ],
)
def _coulomb_sc(tbl_hbm, pairs_hbm, par_hbm, out_hbm,
                idx_v, rows_v, par_v, acc_v, sem):
    wid = lax.axis_index("s") * _NC + lax.axis_index("c")

    pltpu.sync_copy(par_hbm, par_v)
    binv = [par_v[r] for r in range(9)]
    boxm = [par_v[9 + r] for r in range(9)]
    cut2 = par_v[18]
    cutinv = par_v[19]

    lane = lax.iota(jnp.int32, _L)
    col = [jnp.full((_L,), k, jnp.int32) for k in range(4)]
    half = jnp.float32(0.5)
    one = jnp.float32(1.0)
    big_thresh = jnp.float32(8388608.0)

    def floor_f32(t):
        tf = t.astype(jnp.int32).astype(jnp.float32)
        fl = jnp.where(tf > t, tf - one, tf)
        return jnp.where(jnp.abs(t) >= big_thresh, t, fl)

    def chunk_body(c, acc):
        row0 = wid * _ROWS_PER_W + c * _R
        pltpu.sync_copy(pairs_hbm.at[pl.ds(row0, _R)], idx_v)
        copies = [
            pltpu.async_copy(tbl_hbm.at[idx_v.at[j]],
                             rows_v.at[pl.ds(j * _IDXW, _IDXW)], sem)
            for j in range(_R)
        ]
        for cp in copies:
            cp.wait()

        def step(s, acc):
            r_src = (s * _L + lane) * 2
            r_dst = r_src + 1
            sx = plsc.load_gather(rows_v, [r_src, col[0]])
            sy = plsc.load_gather(rows_v, [r_src, col[1]])
            sz = plsc.load_gather(rows_v, [r_src, col[2]])
            sq = plsc.load_gather(rows_v, [r_src, col[3]])
            dx = plsc.load_gather(rows_v, [r_dst, col[0]])
            dy = plsc.load_gather(rows_v, [r_dst, col[1]])
            dz = plsc.load_gather(rows_v, [r_dst, col[2]])
            dq = plsc.load_gather(rows_v, [r_dst, col[3]])

            drx = sx - dx
            dry = sy - dy
            drz = sz - dz
            dsx = drx * binv[0] + dry * binv[3] + drz * binv[6]
            dsy = drx * binv[1] + dry * binv[4] + drz * binv[7]
            dsz = drx * binv[2] + dry * binv[5] + drz * binv[8]
            wx = dsx - floor_f32(dsx + half)
            wy = dsy - floor_f32(dsy + half)
            wz = dsz - floor_f32(dsz + half)
            px = wx * boxm[0] + wy * boxm[3] + wz * boxm[6]
            py = wx * boxm[1] + wy * boxm[4] + wz * boxm[7]
            pz = wx * boxm[2] + wy * boxm[5] + wz * boxm[8]
            d2 = px * px + py * py + pz * pz

            yi = 0x5F3759DF - (plsc.bitcast(d2, jnp.int32) >> 1)
            y = plsc.bitcast(yi, jnp.float32)
            y = y * (jnp.float32(1.5) - half * d2 * y * y)
            y = y * (jnp.float32(1.5) - half * d2 * y * y)
            y = y * (jnp.float32(1.5) - half * d2 * y * y)

            ene = sq * dq * (y - cutinv)
            ene = jnp.where(d2 <= cut2, ene, jnp.float32(0.0))
            return acc + ene

        return lax.fori_loop(0, _STEPS, step, acc)

    acc = lax.fori_loop(0, _CHUNKS, chunk_body,
                        jnp.zeros((_L,), jnp.float32))
    acc_v[...] = acc
    pltpu.sync_copy(acc_v, out_hbm.at[pl.ds(wid * _L, _L)])


def kernel(coords, pairs, box, charges, prefac, cutoff):
    boxinv = jnp.linalg.inv(box)
    tbl = jnp.concatenate(
        [coords, charges[:, None], jnp.zeros((_N, 4), jnp.float32)], axis=1)
    pairs2d = pairs.reshape(-1).astype(jnp.int32).reshape(_IDX_ROWS, _IDXW)
    par = jnp.zeros((20, 16), jnp.float32)
    par = par.at[0:9].set(jnp.broadcast_to(boxinv.reshape(9)[:, None], (9, 16)))
    par = par.at[9:18].set(jnp.broadcast_to(box.reshape(9)[:, None], (9, 16)))
    par = par.at[18].set(jnp.broadcast_to(cutoff * cutoff, (16,)))
    par = par.at[19].set(jnp.broadcast_to(1.0 / cutoff, (16,)))
    partials = _coulomb_sc(tbl, pairs2d, par)
    return jnp.sum(partials) * prefac
